# Initial kernel scaffold; baseline (speedup 1.0000x reference)
import jax
import jax.numpy as jnp
from jax.experimental import pallas as pl


def kernel(T, L, D, avg_f, conductivity, time_step, edge_index):
    raise NotImplementedError("write your pallas kernel here")


# speedup vs baseline: 214.8990x; 214.8990x over previous
import functools

import jax
import jax.numpy as jnp
import numpy as np
from jax import lax
from jax.experimental import pallas as pl
from jax.experimental.pallas import tpu as pltpu
from jax.experimental.pallas import tpu_sc as plsc

N = 100000
E = 6400000

NC = 2
NS = 16
LANES = 16

N_PAD = 100352
NPT = N_PAD // NS
E_PAD = 6553600
EPT = E_PAD // (NC * NS)
CHUNK = 2048
ROWS = CHUNK // 128
N_CHUNKS = EPT // CHUNK


def _sc_kernel(t_hbm, l_hbm, d_hbm, f_hbm, cond_hbm, ts_hbm, src_hbm,
               dst_hbm, part_hbm,
               t_s, w_s, acc_s,
               node_a, node_b, node_c, node_d,
               src_v, dst_v, cond_v, tsrc_v, tdst_v, wsrc_v, en_v, ts_v):
    c = lax.axis_index("c")
    s = lax.axis_index("s")
    wid = c * NS + s

    nb = s * NPT
    pltpu.sync_copy(l_hbm.at[pl.ds(nb, NPT)], node_a)
    pltpu.sync_copy(d_hbm.at[pl.ds(nb, NPT)], node_b)
    pltpu.sync_copy(f_hbm.at[pl.ds(nb, NPT)], node_c)
    pltpu.sync_copy(ts_hbm, ts_v)

    @pl.loop(0, NPT // LANES)
    def _(i):
        sl = pl.ds(i * LANES, LANES)
        node_d[sl] = node_a[sl] * node_b[sl] * node_c[sl]
        node_a[sl] = jnp.zeros((LANES,), jnp.float32)
    pltpu.sync_copy(node_d, w_s.at[pl.ds(nb, NPT)])
    pltpu.sync_copy(node_a, acc_s.at[pl.ds(nb, NPT)])
    pltpu.sync_copy(t_hbm.at[pl.ds(nb, NPT)], node_b)
    pltpu.sync_copy(node_b, t_s.at[pl.ds(nb, NPT)])
    ts_v[...] = ts_v[...] * jnp.float32(0.5 * np.pi)
    plsc.subcore_barrier()

    row_base = wid * (EPT // 128)
    edge_base = wid * EPT

    @pl.loop(0, N_CHUNKS)
    def _(g):
        row = row_base + g * ROWS
        pltpu.sync_copy(src_hbm.at[pl.ds(row, ROWS)], src_v)
        pltpu.sync_copy(dst_hbm.at[pl.ds(row, ROWS)], dst_v)
        pltpu.sync_copy(cond_hbm.at[pl.ds(edge_base + g * CHUNK, CHUNK)],
                        cond_v)
        for j in range(ROWS):
            dsl = pl.ds(j * 128, 128)
            pltpu.sync_copy(t_s.at[src_v.at[j]], tsrc_v.at[dsl])
            pltpu.sync_copy(t_s.at[dst_v.at[j]], tdst_v.at[dsl])
            pltpu.sync_copy(w_s.at[src_v.at[j]], wsrc_v.at[dsl])

        tsk = ts_v[...]

        @pl.loop(0, CHUNK // LANES)
        def _(i):
            sl = pl.ds(i * LANES, LANES)
            ts = tsrc_v[sl]
            td = tdst_v[sl]
            d_t = jnp.maximum(ts - td, jnp.float32(0.0))
            en_v[sl] = (d_t * cond_v[sl]) * (wsrc_v[sl] * (td * td * td)) * tsk

        for j in range(ROWS):
            dsl = pl.ds(j * 128, 128)
            pltpu.sync_copy(en_v.at[dsl], acc_s.at[dst_v.at[j]], add=True)

    plsc.subcore_barrier()

    pltpu.sync_copy(acc_s.at[pl.ds(nb, NPT)], node_a)
    pltpu.sync_copy(node_a, part_hbm.at[c, s])


@jax.jit
def _run_sc(t_p, l_p, d_p, f_p, cond_p, ts16, src2d, dst2d):
    mesh = plsc.VectorSubcoreMesh(core_axis_name="c", subcore_axis_name="s")
    fn = pl.kernel(
        _sc_kernel,
        out_type=jax.ShapeDtypeStruct((NC, NS, NPT), jnp.float32),
        mesh=mesh,
        scratch_types=[
            pltpu.VMEM_SHARED((N_PAD,), jnp.float32),
            pltpu.VMEM_SHARED((N_PAD,), jnp.float32),
            pltpu.VMEM_SHARED((N_PAD,), jnp.float32),
            pltpu.VMEM((NPT,), jnp.float32),
            pltpu.VMEM((NPT,), jnp.float32),
            pltpu.VMEM((NPT,), jnp.float32),
            pltpu.VMEM((NPT,), jnp.float32),
            pltpu.VMEM((ROWS, 128), jnp.int32),
            pltpu.VMEM((ROWS, 128), jnp.int32),
            pltpu.VMEM((CHUNK,), jnp.float32),
            pltpu.VMEM((CHUNK,), jnp.float32),
            pltpu.VMEM((CHUNK,), jnp.float32),
            pltpu.VMEM((CHUNK,), jnp.float32),
            pltpu.VMEM((CHUNK,), jnp.float32),
            pltpu.VMEM((LANES,), jnp.float32),
        ],
    )
    return fn(t_p, l_p, d_p, f_p, cond_p, ts16, src2d, dst2d)


def _combine_body(p_ref, o_ref):
    o_ref[...] = p_ref[0] + p_ref[1]


@jax.jit
def _combine(part):
    p3 = part.reshape(2, N_PAD // 128, 128)
    out = pl.pallas_call(
        _combine_body,
        out_shape=jax.ShapeDtypeStruct((N_PAD // 128, 128), jnp.float32),
    )(p3)
    return out.reshape(N_PAD)[:N]


def kernel(T, L, D, avg_f, conductivity, time_step, edge_index):
    t_p = jnp.pad(T, (0, N_PAD - N))
    l_p = jnp.pad(L, (0, N_PAD - N))
    d_p = jnp.pad(D, (0, N_PAD - N))
    f_p = jnp.pad(avg_f, (0, N_PAD - N))
    cond_p = jnp.pad(conductivity.astype(jnp.float32), (0, E_PAD - E))
    src2d = jnp.pad(edge_index[0], (0, E_PAD - E)).reshape(E_PAD // 128, 128)
    dst2d = jnp.pad(edge_index[1], (0, E_PAD - E)).reshape(E_PAD // 128, 128)
    ts16 = jnp.broadcast_to(time_step.astype(jnp.float32), (LANES,))
    part = _run_sc(t_p, l_p, d_p, f_p, cond_p, ts16, src2d, dst2d)
    return _combine(part)
